# baseline (speedup 1.0000x reference)
import functools

import jax
import jax.numpy as jnp
from jax import lax
from jax.experimental import pallas as pl
from jax.experimental.pallas import tpu as pltpu
from jax.experimental.pallas import tpu_sc as plsc

EMBED = 4096
EXPERTS = 64
K = 8
TOKENS = 16384


_BT = 512


def _logits_body(x_ref, w_ref, b_ref, o_ref):
    o_ref[...] = (
        jnp.dot(x_ref[...], w_ref[...], preferred_element_type=jnp.float32)
        + b_ref[...]
    )


def _logits(x2d, W, b2d):
    nt = x2d.shape[0]
    return pl.pallas_call(
        _logits_body,
        grid=(nt // _BT,),
        in_specs=[
            pl.BlockSpec((_BT, EMBED), lambda i: (i, 0)),
            pl.BlockSpec((EMBED, EXPERTS), lambda i: (0, 0)),
            pl.BlockSpec((1, EXPERTS), lambda i: (0, 0)),
        ],
        out_specs=pl.BlockSpec((_BT, EXPERTS), lambda i: (i, 0)),
        out_shape=jax.ShapeDtypeStruct((nt, EXPERTS), jnp.float32),
    )(x2d, W, b2d)


_NCHUNK = 1
_CT = TOKENS // _NCHUNK
_NCORES = 1
_NW = 16 * _NCORES
_TPW = _CT // _NW


def _sc_topk_body(lg_hbm, ov_hbm, oi_hbm, lg, ov, oi, sem):
    wid = lax.axis_index("s") * _NCORES + lax.axis_index("c")
    base_tok = wid * _TPW

    pltpu.sync_copy(lg_hbm.at[pl.ds(base_tok * EXPERTS, _TPW * EXPERTS)], lg)

    lane = lax.iota(jnp.int32, 16)
    lo8 = lane < 8
    lane_st = jnp.where(lo8, lane, 0)
    pay = [lane + 16 * q for q in range(4)]

    def merge(ak, ap, bk, bp):
        ck = jnp.where(lo8, ak, lax.rev(bk, (0,)))
        cp = jnp.where(lo8, ap, lax.rev(bp, (0,)))
        return plsc.sort_key_val(ck, cp, descending=True)

    @plsc.parallel_loop(0, _TPW, 1, unroll=4)
    def token(t):
        base = t * EXPERTS
        s0, q0 = plsc.sort_key_val(lg[pl.ds(base, 16)], pay[0], descending=True)
        s1, q1 = plsc.sort_key_val(
            lg[pl.ds(base + 16, 16)], pay[1], descending=True
        )
        s2, q2 = plsc.sort_key_val(
            lg[pl.ds(base + 32, 16)], pay[2], descending=True
        )
        s3, q3 = plsc.sort_key_val(
            lg[pl.ds(base + 48, 16)], pay[3], descending=True
        )
        ka, qa = merge(s0, q0, s1, q1)
        kb, qb = merge(s2, q2, s3, q3)
        kf, qf = merge(ka, qa, kb, qb)
        ex = jnp.exp(jnp.minimum(kf, 80.0))
        em = jnp.where(lo8, ex, 0.0)
        r = em / jnp.broadcast_to(jnp.sum(em), (16,))
        idxs = t * K + lane_st
        plsc.store_scatter(ov, [idxs], r, mask=lo8)
        plsc.store_scatter(oi, [idxs], qf, mask=lo8)

    pltpu.sync_copy(ov, ov_hbm.at[pl.ds(base_tok * K, _TPW * K)])
    pltpu.sync_copy(oi, oi_hbm.at[pl.ds(base_tok * K, _TPW * K)])


@functools.partial(
    pl.kernel,
    mesh=plsc.VectorSubcoreMesh(
        core_axis_name="c", subcore_axis_name="s", num_cores=_NCORES
    ),
    out_type=[
        jax.ShapeDtypeStruct((_CT * K,), jnp.float32),
        jax.ShapeDtypeStruct((_CT * K,), jnp.int32),
    ],
    scratch_types=[
        pltpu.VMEM((_TPW * EXPERTS,), jnp.float32),
        pltpu.VMEM((_TPW * K,), jnp.float32),
        pltpu.VMEM((_TPW * K,), jnp.int32),
        pltpu.SemaphoreType.DMA,
    ],
    compiler_params=pltpu.CompilerParams(needs_layout_passes=False),
)
def _sc_topk(lg_hbm, ov_hbm, oi_hbm, lg, ov, oi, sem):
    _sc_topk_body(lg_hbm, ov_hbm, oi_hbm, lg, ov, oi, sem)


def kernel(inputs, W, b):
    B, S, E = inputs.shape
    x2d = inputs.reshape(B * S, E)
    b2d = b.reshape(1, EXPERTS)
    vals_parts = []
    idx_parts = []
    for c in range(_NCHUNK):
        lg = _logits(x2d[c * _CT : (c + 1) * _CT], W, b2d)
        v, i = _sc_topk(lg.reshape(-1))
        vals_parts.append(v)
        idx_parts.append(i)
    vals = jnp.concatenate(vals_parts)
    idx = jnp.concatenate(idx_parts)
    return vals.reshape(B, S, K), idx.reshape(B, S, K)


# speedup vs baseline: 1.0437x; 1.0437x over previous
import functools

import jax
import jax.numpy as jnp
from jax import lax
from jax.experimental import pallas as pl
from jax.experimental.pallas import tpu as pltpu
from jax.experimental.pallas import tpu_sc as plsc

EMBED = 4096
EXPERTS = 64
EPAD = 128
K = 8
TOKENS = 16384


_BT = 512


def _logits_body(x_ref, w_ref, b_ref, o_ref):
    res = (
        jnp.dot(x_ref[...], w_ref[...], preferred_element_type=jnp.float32)
        + b_ref[...]
    )
    o_ref[...] = jnp.concatenate(
        [res, jnp.full((_BT, EPAD - EXPERTS), -1e30, jnp.float32)], axis=1
    )


def _logits(x2d, Wp, bp):
    nt = x2d.shape[0]
    return pl.pallas_call(
        _logits_body,
        grid=(nt // _BT,),
        in_specs=[
            pl.BlockSpec((_BT, EMBED), lambda i: (i, 0)),
            pl.BlockSpec((EMBED, EXPERTS), lambda i: (0, 0)),
            pl.BlockSpec((1, EXPERTS), lambda i: (0, 0)),
        ],
        out_specs=pl.BlockSpec((_BT, EPAD), lambda i: (i, 0)),
        out_shape=jax.ShapeDtypeStruct((nt, EPAD), jnp.float32),
    )(x2d, Wp, bp)


_NCORES = 2
_NW = 16 * _NCORES
_TPW = TOKENS // _NW
_TB = 128


def _sc_topk_body(lg_hbm, ov_hbm, oi_hbm, lg, ov, oi, sem):
    wid = lax.axis_index("s") * _NCORES + lax.axis_index("c")
    base_tok = wid * _TPW

    lane = lax.iota(jnp.int32, 16)
    lo8 = lane < 8
    lane_st = jnp.where(lo8, lane, 0)
    pay = [lane + 16 * q for q in range(4)]

    def merge(ak, ap, bk, bp):
        ck = jnp.where(lo8, ak, lax.rev(bk, (0,)))
        cp = jnp.where(lo8, ap, lax.rev(bp, (0,)))
        return plsc.sort_key_val(ck, cp, descending=True)

    for blk in range(_TPW // _TB):
        pltpu.sync_copy(lg_hbm.at[pl.ds(base_tok + blk * _TB, _TB), :], lg)
        out0 = blk * _TB

        @plsc.parallel_loop(0, _TB, 1, unroll=4)
        def token(t):
            s0, q0 = plsc.sort_key_val(
                lg[t, pl.ds(0, 16)], pay[0], descending=True
            )
            s1, q1 = plsc.sort_key_val(
                lg[t, pl.ds(16, 16)], pay[1], descending=True
            )
            s2, q2 = plsc.sort_key_val(
                lg[t, pl.ds(32, 16)], pay[2], descending=True
            )
            s3, q3 = plsc.sort_key_val(
                lg[t, pl.ds(48, 16)], pay[3], descending=True
            )
            ka, qa = merge(s0, q0, s1, q1)
            kb, qb = merge(s2, q2, s3, q3)
            kf, qf = merge(ka, qa, kb, qb)
            ex = jnp.exp(jnp.minimum(kf, 80.0))
            em = jnp.where(lo8, ex, 0.0)
            r = em / jnp.broadcast_to(jnp.sum(em), (16,))
            idxs = (out0 + t) * K + lane_st
            plsc.store_scatter(ov, [idxs], r, mask=lo8)
            plsc.store_scatter(oi, [idxs], qf, mask=lo8)

    pltpu.sync_copy(ov, ov_hbm.at[pl.ds(base_tok * K, _TPW * K)])
    pltpu.sync_copy(oi, oi_hbm.at[pl.ds(base_tok * K, _TPW * K)])


@functools.partial(
    pl.kernel,
    mesh=plsc.VectorSubcoreMesh(
        core_axis_name="c", subcore_axis_name="s", num_cores=_NCORES
    ),
    out_type=[
        jax.ShapeDtypeStruct((TOKENS * K,), jnp.float32),
        jax.ShapeDtypeStruct((TOKENS * K,), jnp.int32),
    ],
    scratch_types=[
        pltpu.VMEM((_TB, EPAD), jnp.float32),
        pltpu.VMEM((_TPW * K,), jnp.float32),
        pltpu.VMEM((_TPW * K,), jnp.int32),
        pltpu.SemaphoreType.DMA,
    ],
    compiler_params=pltpu.CompilerParams(needs_layout_passes=False),
)
def _sc_topk(lg_hbm, ov_hbm, oi_hbm, lg, ov, oi, sem):
    _sc_topk_body(lg_hbm, ov_hbm, oi_hbm, lg, ov, oi, sem)


def kernel(inputs, W, b):
    B, S, E = inputs.shape
    x2d = inputs.reshape(B * S, E)
    logits = _logits(x2d, W, b.reshape(1, EXPERTS))
    vals, idx = _sc_topk(logits)
    return vals.reshape(B, S, K), idx.reshape(B, S, K)
